# baseline (speedup 1.0000x reference)
import functools

import jax
import jax.numpy as jnp
from jax import lax
from jax.experimental import pallas as pl
from jax.experimental.pallas import tpu as pltpu
from jax.experimental.pallas import tpu_sc as plsc

_N_CLS = 1000
_N_ENV = 128
_BATCH = 16384

_info = plsc.get_sparse_core_info()
_NC, _NS = _info.num_cores, _info.num_subcores
_NW = _NC * _NS
_BPW = _BATCH // _NW


def _softmax_body(x_ref, o_ref):
    x = x_ref[...]
    m = jnp.max(x, axis=-1, keepdims=True)
    e = jnp.exp(x - m)
    o_ref[...] = e / jnp.sum(e, axis=-1, keepdims=True)


def _softmax_table(mat):
    return pl.pallas_call(
        _softmax_body,
        out_shape=jax.ShapeDtypeStruct(mat.shape, mat.dtype),
    )(mat)


_mesh = plsc.VectorSubcoreMesh(core_axis_name="c", subcore_axis_name="s")

_CH = 64
_NCHUNK = _BPW // _CH


@functools.partial(
    pl.kernel,
    mesh=_mesh,
    out_type=jax.ShapeDtypeStruct((_BATCH, _N_ENV), jnp.float32),
    scratch_types=[
        pltpu.VMEM((_BPW,), jnp.int32),
        pltpu.VMEM((_NCHUNK, _CH, _N_ENV), jnp.float32),
        pltpu.SemaphoreType.DMA,
        pltpu.SemaphoreType.DMA,
    ],
)
def _gather_sc(table_hbm, idx_hbm, out_hbm, idx_v, buf, gsem, wsem):
    wid = lax.axis_index("s") * _NC + lax.axis_index("c")
    base = wid * _BPW
    pltpu.sync_copy(idx_hbm.at[pl.ds(base, _BPW)], idx_v)
    gathers = [
        pltpu.async_copy(
            table_hbm.at[idx_v.at[pl.ds(k * _CH, _CH)]], buf.at[k], gsem
        )
        for k in range(_NCHUNK)
    ]
    writes = []
    for k in range(_NCHUNK):
        gathers[k].wait()
        writes.append(
            pltpu.async_copy(
                buf.at[k], out_hbm.at[pl.ds(base + k * _CH, _CH)], wsem
            )
        )
    for w in writes:
        w.wait()


def kernel(label, partition_matrix):
    sm = _softmax_table(partition_matrix)
    return _gather_sc(sm, label.astype(jnp.int32))


# speedup vs baseline: 1.0200x; 1.0200x over previous
import functools

import jax
import jax.numpy as jnp
from jax import lax
from jax.experimental import pallas as pl
from jax.experimental.pallas import tpu as pltpu
from jax.experimental.pallas import tpu_sc as plsc

_N_CLS = 1000
_N_ENV = 128
_BATCH = 16384
_LANES = 128 // 16

_info = plsc.get_sparse_core_info()
_NC, _NS = _info.num_cores, _info.num_subcores
_NW = _NC * _NS
_BPW = _BATCH // _NW

_RPT = 64
_LAST_ROW0 = _N_CLS - _RPT

_mesh = plsc.VectorSubcoreMesh(core_axis_name="c", subcore_axis_name="s")


@functools.partial(
    pl.kernel,
    mesh=_mesh,
    out_type=jax.ShapeDtypeStruct((_BATCH, _N_ENV), jnp.float32),
    scratch_types=[
        pltpu.VMEM((_RPT, _N_ENV), jnp.float32),
        pltpu.VMEM((_BPW,), jnp.int32),
        pltpu.VMEM((_BPW, _N_ENV), jnp.float32),
        pltpu.VMEM_SHARED((_N_CLS, _N_ENV), jnp.float32),
        pltpu.SemaphoreType.DMA,
        pltpu.SemaphoreType.DMA,
    ],
)
def _partition_sc(table_hbm, idx_hbm, out_hbm, chunk, idx_v, rows_v, shared,
                  isem, gsem):
    c = lax.axis_index("c")
    s = lax.axis_index("s")
    wid = s * _NC + c
    base = wid * _BPW

    idx_cp = pltpu.async_copy(idx_hbm.at[pl.ds(base, _BPW)], idx_v, isem)

    row0 = jnp.minimum(s * _RPT, _LAST_ROW0)
    pltpu.sync_copy(table_hbm.at[pl.ds(row0, _RPT)], chunk)

    lanes = lax.iota(jnp.int32, 16)
    _dnums = lax.GatherDimensionNumbers(
        offset_dims=(), collapsed_slice_dims=(0,), start_index_map=(0,))

    def _shuffle(v, idx):
        return lax.gather(v, idx[:, None], _dnums, slice_sizes=(1,),
                          mode=lax.GatherScatterMode.PROMISE_IN_BOUNDS)

    def row_fn(r, carry):
        vs = [chunk[r, pl.ds(j * 16, 16)] for j in range(_LANES)]
        m = vs[0]
        for v in vs[1:]:
            m = jnp.maximum(m, v)
        for k in (8, 4, 2, 1):
            m = jnp.maximum(m, _shuffle(m, lanes ^ k))
        es = [jnp.exp(v - m) for v in vs]
        t = es[0]
        for e in es[1:]:
            t = t + e
        for k in (8, 4, 2, 1):
            t = t + _shuffle(t, lanes ^ k)
        inv = 1.0 / t
        for j in range(_LANES):
            chunk[r, pl.ds(j * 16, 16)] = es[j] * inv
        return carry

    lax.fori_loop(0, _RPT, row_fn, 0)

    pltpu.sync_copy(chunk, shared.at[pl.ds(row0, _RPT)])
    plsc.subcore_barrier()

    idx_cp.wait()
    pltpu.async_copy(shared.at[idx_v], rows_v, gsem).wait()
    pltpu.sync_copy(rows_v, out_hbm.at[pl.ds(base, _BPW)])


def kernel(label, partition_matrix):
    return _partition_sc(partition_matrix, label.astype(jnp.int32))


# speedup vs baseline: 1.2155x; 1.1917x over previous
import functools

import jax
import jax.numpy as jnp
from jax import lax
from jax.experimental import pallas as pl
from jax.experimental.pallas import tpu as pltpu
from jax.experimental.pallas import tpu_sc as plsc

_N_CLS = 1000
_N_ENV = 128
_BATCH = 16384

_info = plsc.get_sparse_core_info()
_NC, _NS = _info.num_cores, _info.num_subcores
_NW = _NC * _NS
_BPW = _BATCH // _NW

_RPT = 64
_LAST_ROW0 = _N_CLS - _RPT

_CH = 128
_NCHUNK = _BPW // _CH


def _softmax_body(x_ref, o_ref):
    x = x_ref[...]
    m = jnp.max(x, axis=-1, keepdims=True)
    e = jnp.exp(x - m)
    o_ref[...] = e / jnp.sum(e, axis=-1, keepdims=True)


def _softmax_table(mat):
    return pl.pallas_call(
        _softmax_body,
        out_shape=jax.ShapeDtypeStruct(mat.shape, mat.dtype),
    )(mat)


_mesh = plsc.VectorSubcoreMesh(core_axis_name="c", subcore_axis_name="s")


@functools.partial(
    pl.kernel,
    mesh=_mesh,
    out_type=jax.ShapeDtypeStruct((_BATCH, _N_ENV), jnp.float32),
    scratch_types=[
        pltpu.VMEM((_BPW,), jnp.int32),
        pltpu.VMEM((_NCHUNK, _CH, _N_ENV), jnp.float32),
        pltpu.VMEM_SHARED((_N_CLS, _N_ENV), jnp.float32),
        pltpu.SemaphoreType.DMA,
        pltpu.SemaphoreType.DMA,
        pltpu.SemaphoreType.DMA,
    ],
)
def _gather_sc(table_hbm, idx_hbm, out_hbm, idx_v, buf, shared, isem, gsem,
               wsem):
    c = lax.axis_index("c")
    s = lax.axis_index("s")
    wid = s * _NC + c
    base = wid * _BPW

    idx_cp = pltpu.async_copy(idx_hbm.at[pl.ds(base, _BPW)], idx_v, isem)

    row0 = jnp.minimum(s * _RPT, _LAST_ROW0)
    pltpu.sync_copy(table_hbm.at[pl.ds(row0, _RPT)],
                    shared.at[pl.ds(row0, _RPT)])
    plsc.subcore_barrier()
    idx_cp.wait()

    gathers = [
        pltpu.async_copy(shared.at[idx_v.at[pl.ds(k * _CH, _CH)]], buf.at[k],
                         gsem)
        for k in range(_NCHUNK)
    ]
    writes = []
    for k in range(_NCHUNK):
        gathers[k].wait()
        writes.append(
            pltpu.async_copy(buf.at[k], out_hbm.at[pl.ds(base + k * _CH, _CH)],
                             wsem))
    for w in writes:
        w.wait()


def kernel(label, partition_matrix):
    sm = _softmax_table(partition_matrix)
    return _gather_sc(sm, label.astype(jnp.int32))


# speedup vs baseline: 1.2203x; 1.0039x over previous
import functools

import jax
import jax.numpy as jnp
from jax import lax
from jax.experimental import pallas as pl
from jax.experimental.pallas import tpu as pltpu
from jax.experimental.pallas import tpu_sc as plsc

_N_CLS = 1000
_N_ENV = 128
_BATCH = 16384

_info = plsc.get_sparse_core_info()
_NC, _NS = _info.num_cores, _info.num_subcores
_NW = _NC * _NS
_BPW = _BATCH // _NW

_RPT = 64
_LAST_ROW0 = _N_CLS - _RPT

_CH = 128
_NCHUNK = _BPW // _CH


def _softmax_body(x_ref, o_ref):
    x = x_ref[...]
    m = jnp.max(x, axis=-1, keepdims=True)
    e = jnp.exp(x - m)
    o_ref[...] = e * (1.0 / jnp.sum(e, axis=-1, keepdims=True))


def _softmax_table(mat):
    return pl.pallas_call(
        _softmax_body,
        out_shape=jax.ShapeDtypeStruct(mat.shape, mat.dtype),
    )(mat)


_mesh = plsc.VectorSubcoreMesh(core_axis_name="c", subcore_axis_name="s")


@functools.partial(
    pl.kernel,
    mesh=_mesh,
    out_type=jax.ShapeDtypeStruct((_BATCH, _N_ENV), jnp.float32),
    scratch_types=[
        pltpu.VMEM((_BPW,), jnp.int32),
        pltpu.VMEM((_NCHUNK, _CH, _N_ENV), jnp.float32),
        pltpu.VMEM_SHARED((_N_CLS, _N_ENV), jnp.float32),
        pltpu.SemaphoreType.DMA,
        pltpu.SemaphoreType.DMA,
        pltpu.SemaphoreType.DMA,
    ],
)
def _gather_sc(table_hbm, idx_hbm, out_hbm, idx_v, buf, shared, isem, gsem,
               wsem):
    c = lax.axis_index("c")
    s = lax.axis_index("s")
    wid = s * _NC + c
    base = wid * _BPW

    idx_cp = pltpu.async_copy(idx_hbm.at[pl.ds(base, _BPW)], idx_v, isem)

    row0 = jnp.minimum(s * _RPT, _LAST_ROW0)
    pltpu.sync_copy(table_hbm.at[pl.ds(row0, _RPT)],
                    shared.at[pl.ds(row0, _RPT)])
    plsc.subcore_barrier()
    idx_cp.wait()

    gathers = [
        pltpu.async_copy(shared.at[idx_v.at[pl.ds(k * _CH, _CH)]], buf.at[k],
                         gsem)
        for k in range(_NCHUNK)
    ]
    writes = []
    for k in range(_NCHUNK):
        gathers[k].wait()
        writes.append(
            pltpu.async_copy(buf.at[k], out_hbm.at[pl.ds(base + k * _CH, _CH)],
                             wsem))
    for w in writes:
        w.wait()


def kernel(label, partition_matrix):
    sm = _softmax_table(partition_matrix)
    return _gather_sc(sm, label.astype(jnp.int32))
